# baseline (speedup 1.0000x reference)
import functools

import jax
import jax.numpy as jnp
from jax import lax
from jax.experimental import pallas as pl
from jax.experimental.pallas import tpu as pltpu
from jax.experimental.pallas import tpu_sc as plsc

N = 4096
D = 128
H = 64
K = 16

NC = 2
NS = 16
NW = NC * NS
L = 16

PAIRS = N * K
PPW = PAIRS // NW
CH = 256
ROWS_PW = N // NW
G = 8
NGROUPS = ROWS_PW // G


def _prep_T_body(x_ref, t_ref, e_ref, Wn1b_ref, bn1_ref, T_ref):
    x = x_ref[...]
    Bm = (jnp.dot(x, Wn1b_ref[...], preferred_element_type=jnp.float32)
          + bn1_ref[...])
    r = t_ref[...] - e_ref[...]
    T_ref[...] = jnp.concatenate(
        [Bm, r, jnp.zeros((x.shape[0], D - H - 1), jnp.float32)], axis=1)


def _prep_T(x, t, e_hat, Wn1b, bn1):
    return pl.pallas_call(
        _prep_T_body,
        out_shape=jax.ShapeDtypeStruct((N, D), jnp.float32),
    )(x, t.reshape(N, 1), e_hat.reshape(N, 1), Wn1b, bn1.reshape(1, H))


def _prep_rest_body(x_ref, t_ref, e_ref, Wn1a_ref,
                    Ws1_ref, bs1_ref, Ws2_ref, bs2_ref, Ws3_ref, bs3_ref,
                    A_ref, g_ref, sc_ref):
    x = x_ref[...]
    A = jnp.dot(x, Wn1a_ref[...], preferred_element_type=jnp.float32)
    A_ref[...] = jnp.concatenate([A, A], axis=1)
    r = t_ref[...] - e_ref[...]
    h = jax.nn.relu(jnp.dot(x, Ws1_ref[...], preferred_element_type=jnp.float32)
                    + bs1_ref[...])
    h = jax.nn.relu(jnp.dot(h, Ws2_ref[...], preferred_element_type=jnp.float32)
                    + bs2_ref[...])
    g = jnp.sum(h * Ws3_ref[...].reshape(1, H), axis=1, keepdims=True) + bs3_ref[0, 0]
    g_ref[...] = g
    sc_ref[...] = g * r


def _tc_prep_rest(x, t, e_hat, Wn1a, Ws1, bs1, Ws2, bs2, Ws3, bs3):
    out_shapes = (
        jax.ShapeDtypeStruct((N, D), jnp.float32),
        jax.ShapeDtypeStruct((N, 1), jnp.float32),
        jax.ShapeDtypeStruct((N, 1), jnp.float32),
    )
    return pl.pallas_call(_prep_rest_body, out_shape=out_shapes)(
        x, t.reshape(N, 1), e_hat.reshape(N, 1), Wn1a,
        Ws1, bs1.reshape(1, H), Ws2, bs2.reshape(1, H), Ws3, bs3.reshape(1, 1))


def _sc_gather_body(T_hbm, idx_hbm, Tg_hbm,
                    idx_v0, idx_v1, rows_v0, rows_v1, pk_v0, pk_v1,
                    gsem0, gsem1, osem0, osem1):
    wid = lax.axis_index("s") * NC + lax.axis_index("c")
    ppw = idx_hbm.shape[0] // NW
    nch = ppw // CH
    base = pl.multiple_of(wid * ppw, ppw)
    idxs = (idx_v0, idx_v1)
    rows = (rows_v0, rows_v1)
    pks = (pk_v0, pk_v1)
    gsems = (gsem0, gsem1)
    osems = (osem0, osem1)
    gh = [None] * nch
    oh = [None] * nch

    def _finish(j):
        bj = j % 2
        gh[j].wait()
        if j >= 2:
            oh[j - 2].wait()
        rv, pv = rows[bj], pks[bj]

        def _pack(q, carry):
            for hh in range(H // L):
                pv[q, pl.ds(hh * L, L)] = rv[2 * q, pl.ds(hh * L, L)]
                pv[q, pl.ds(H + hh * L, L)] = rv[2 * q + 1, pl.ds(hh * L, L)]
            return carry

        lax.fori_loop(0, CH // 2, _pack, 0)
        ooff = pl.multiple_of((base + j * CH) // 2, CH // 2)
        oh[j] = pltpu.async_copy(pv, Tg_hbm.at[pl.ds(ooff, CH // 2)], osems[bj])

    for c in range(nch):
        b = c % 2
        off = pl.multiple_of(base + c * CH, CH)
        pltpu.sync_copy(idx_hbm.at[pl.ds(off, CH)], idxs[b])
        gh[c] = pltpu.async_copy(T_hbm.at[idxs[b]], rows[b], gsems[b])
        if c >= 1:
            _finish(c - 1)
    _finish(nch - 1)
    oh[nch - 2].wait()
    oh[nch - 1].wait()


def _sc_gather(T, chosen_flat):
    npairs = chosen_flat.shape[0]
    mesh = plsc.VectorSubcoreMesh(core_axis_name="c", subcore_axis_name="s")
    kern = pl.kernel(
        _sc_gather_body,
        out_type=jax.ShapeDtypeStruct((npairs // 2, D), jnp.float32),
        mesh=mesh,
        compiler_params=pltpu.CompilerParams(needs_layout_passes=False),
        scratch_types=[
            pltpu.VMEM((CH,), jnp.int32),
            pltpu.VMEM((CH,), jnp.int32),
            pltpu.VMEM((CH, D), jnp.float32),
            pltpu.VMEM((CH, D), jnp.float32),
            pltpu.VMEM((CH // 2, D), jnp.float32),
            pltpu.VMEM((CH // 2, D), jnp.float32),
            pltpu.SemaphoreType.DMA,
            pltpu.SemaphoreType.DMA,
            pltpu.SemaphoreType.DMA,
            pltpu.SemaphoreType.DMA,
        ],
    )
    return kern(T, chosen_flat)


def _main_body(A2_ref, P_ref, W2d_ref, b2d_ref, w3d_ref, bn3_ref, m_ref):
    RQ = P_ref.shape[0]
    nodes = A2_ref.shape[0]
    P3 = P_ref[...].reshape(nodes, RQ // nodes, D)
    h1 = jax.nn.relu(P3 + A2_ref[...][:, None, :]).reshape(RQ, D)
    h2 = jax.nn.relu(jnp.dot(h1, W2d_ref[...], preferred_element_type=jnp.float32)
                     + b2d_ref[...])
    mm = h2 * w3d_ref[...]
    m0 = jnp.sum(mm[:, :H], axis=1) + bn3_ref[0, 0]
    m1 = jnp.sum(mm[:, H:], axis=1) + bn3_ref[0, 0]
    m_ref[...] = jnp.concatenate([m0.reshape(1, RQ), m1.reshape(1, RQ)], axis=0)


def _tc_main(A2, P, W2d, b2d, w3d, bn3):
    RQ = 4096
    NB = RQ // (K // 2)
    grid = (P.shape[0] // RQ,)
    return pl.pallas_call(
        _main_body,
        grid=grid,
        in_specs=[
            pl.BlockSpec((NB, D), lambda i: (i, 0)),
            pl.BlockSpec((RQ, D), lambda i: (i, 0)),
            pl.BlockSpec((D, D), lambda i: (0, 0)),
            pl.BlockSpec((1, D), lambda i: (0, 0)),
            pl.BlockSpec((1, D), lambda i: (0, 0)),
            pl.BlockSpec((1, 1), lambda i: (0, 0)),
        ],
        out_specs=pl.BlockSpec((2, RQ), lambda i: (0, i)),
        out_shape=jax.ShapeDtypeStruct((2, P.shape[0]), jnp.float32),
    )(A2, P, W2d, b2d, w3d, bn3.reshape(1, 1))


def _sc_scatter_body(idx_hbm, m_hbm, r_hbm, sc_hbm, b_hbm, out_hbm, y_hbm,
                     cidx_v, m_v, vals_v, r_v, sc_v, b_v, y_v,
                     buf0, buf1, sem0, sem1):
    wid = lax.axis_index("s") * NC + lax.axis_index("c")
    base = pl.multiple_of(wid * PPW, PPW)
    rbase = pl.multiple_of(wid * ROWS_PW, ROWS_PW)
    pltpu.sync_copy(idx_hbm.at[pl.ds(base, PPW)], cidx_v)
    pltpu.sync_copy(m_hbm.at[pl.ds(base, PPW)], m_v)
    pltpu.sync_copy(r_hbm, r_v)
    pltpu.sync_copy(sc_hbm.at[pl.ds(rbase, ROWS_PW)], sc_v)
    pltpu.sync_copy(b_hbm, b_v)

    z16 = jnp.zeros((L,), jnp.float32)
    b_vec = b_v[...]
    lane = lax.iota(jnp.int32, L)

    def _soft(o):
        acc = z16
        for rr in range(L):
            sl = pl.ds(o * (L * K) + rr * K, L)
            m = m_v[sl]
            cols = cidx_v[sl]
            am = b_vec * jnp.abs(m)
            e = jnp.exp(am - jnp.max(am, axis=0))
            s = lax.broadcast_in_dim(jnp.sum(e, axis=0), (L,), ())
            vals = m * e / s
            prg = plsc.load_gather(r_v, [cols])
            neigh = jnp.sum(prg * vals, axis=0)
            acc = jnp.where(lane == rr, neigh, acc)
            grow = rbase + o * L + rr
            vals_v[sl] = jnp.where(cols == grow, 0.0, vals)
        y_v[pl.ds(o * L, L)] = acc + sc_v[pl.ds(o * L, L)]

    def _zero(i, carry):
        for rr in range(G):
            buf0[rr, pl.ds(i * L, L)] = z16
            buf1[rr, pl.ds(i * L, L)] = z16
        return carry

    lax.fori_loop(0, N // L, _zero, 0)

    bufs = (buf0, buf1)
    sems = (sem0, sem1)
    rowids = [jnp.full((L,), rr, jnp.int32) for rr in range(G)]
    handles = [None] * NGROUPS
    for g in range(NGROUPS):
        if g % 2 == 0:
            _soft(g // 2)
        buf = bufs[g % 2]
        if g >= 2:
            handles[g - 2].wait()
            for rr in range(G):
                cols = cidx_v[pl.ds(((g - 2) * G + rr) * K, L)]
                plsc.store_scatter(buf, [rowids[rr], cols], z16)
        for rr in range(G):
            cols = cidx_v[pl.ds((g * G + rr) * K, L)]
            v = vals_v[pl.ds((g * G + rr) * K, L)]
            plsc.store_scatter(buf, [rowids[rr], cols], v)
        row0 = pl.multiple_of(wid * ROWS_PW + g * G, G)
        handles[g] = pltpu.async_copy(buf, out_hbm.at[pl.ds(row0, G)],
                                      sems[g % 2])
    pltpu.sync_copy(y_v, y_hbm.at[pl.ds(rbase, ROWS_PW)])
    handles[NGROUPS - 2].wait()
    handles[NGROUPS - 1].wait()


def _sc_scatter(chosen_flat, m_flat, r, self_contrib, b_vec):
    mesh = plsc.VectorSubcoreMesh(core_axis_name="c", subcore_axis_name="s")
    kern = pl.kernel(
        _sc_scatter_body,
        out_type=(
            jax.ShapeDtypeStruct((N, N), jnp.float32),
            jax.ShapeDtypeStruct((N,), jnp.float32),
        ),
        mesh=mesh,
        compiler_params=pltpu.CompilerParams(needs_layout_passes=False),
        scratch_types=[
            pltpu.VMEM((PPW,), jnp.int32),
            pltpu.VMEM((PPW,), jnp.float32),
            pltpu.VMEM((PPW,), jnp.float32),
            pltpu.VMEM((N,), jnp.float32),
            pltpu.VMEM((ROWS_PW,), jnp.float32),
            pltpu.VMEM((L,), jnp.float32),
            pltpu.VMEM((ROWS_PW,), jnp.float32),
            pltpu.VMEM((G, N), jnp.float32),
            pltpu.VMEM((G, N), jnp.float32),
            pltpu.SemaphoreType.DMA,
            pltpu.SemaphoreType.DMA,
        ],
    )
    return kern(chosen_flat, m_flat, r, self_contrib, b_vec)


def kernel(x, t, e_hat, nbrs_idx, Wn1, bn1, Wn2, bn2, Wn3, bn3,
           Ws1, bs1, Ws2, bs2, Ws3, bs3, b):
    chosen_flat = nbrs_idx[:, 1:].reshape(PAIRS)
    T = _prep_T(x, t, e_hat, Wn1[D:], bn1)
    P = _sc_gather(T, chosen_flat)
    A2, g, self_contrib = _tc_prep_rest(
        x, t, e_hat, Wn1[:D], Ws1, bs1, Ws2, bs2, Ws3, bs3)
    W2d = (jnp.zeros((D, D), jnp.float32)
           .at[:H, :H].set(Wn2).at[H:, H:].set(Wn2))
    b2d = jnp.concatenate([bn2, bn2]).reshape(1, D)
    w3d = jnp.tile(Wn3[:, 0], 2).reshape(1, D)
    m01 = _tc_main(A2, P, W2d, b2d, w3d, bn3)
    m_flat = m01.T.reshape(PAIRS)
    r = T[:, H]
    pairwise, ypred = _sc_scatter(
        chosen_flat, m_flat, r, self_contrib.reshape(N),
        jnp.full((L,), b, jnp.float32))
    return ypred, pairwise, g.reshape(N)


# speedup vs baseline: 1.2089x; 1.2089x over previous
import functools

import jax
import jax.numpy as jnp
from jax import lax
from jax.experimental import pallas as pl
from jax.experimental.pallas import tpu as pltpu
from jax.experimental.pallas import tpu_sc as plsc

N = 4096
D = 128
H = 64
K = 16

NC = 2
NS = 16
NW = NC * NS
L = 16

PAIRS = N * K
PPW = PAIRS // NW
CH = 512
ROWS_PW = N // NW
G = 8
NGROUPS = ROWS_PW // G


def _prep_T_body(x_ref, t_ref, e_ref, Wn1b_ref, bn1_ref, T_ref):
    x = x_ref[...]
    Bm = (jnp.dot(x, Wn1b_ref[...], preferred_element_type=jnp.float32)
          + bn1_ref[...])
    r = t_ref[...] - e_ref[...]
    T_ref[...] = jnp.concatenate(
        [Bm, r, jnp.zeros((x.shape[0], D - H - 1), jnp.float32)], axis=1)


def _prep_T(x, t, e_hat, Wn1b, bn1):
    return pl.pallas_call(
        _prep_T_body,
        out_shape=jax.ShapeDtypeStruct((N, D), jnp.float32),
    )(x, t.reshape(N, 1), e_hat.reshape(N, 1), Wn1b, bn1.reshape(1, H))


def _prep_rest_body(x_ref, t_ref, e_ref, Wn1a_ref,
                    Ws1_ref, bs1_ref, Ws2_ref, bs2_ref, Ws3_ref, bs3_ref,
                    A_ref, g_ref, sc_ref):
    x = x_ref[...]
    A_ref[...] = jnp.dot(x, Wn1a_ref[...], preferred_element_type=jnp.float32)
    r = t_ref[...] - e_ref[...]
    h = jax.nn.relu(jnp.dot(x, Ws1_ref[...], preferred_element_type=jnp.float32)
                    + bs1_ref[...])
    h = jax.nn.relu(jnp.dot(h, Ws2_ref[...], preferred_element_type=jnp.float32)
                    + bs2_ref[...])
    g = jnp.sum(h * Ws3_ref[...].reshape(1, H), axis=1, keepdims=True) + bs3_ref[0, 0]
    g_ref[...] = g
    sc_ref[...] = g * r


def _tc_prep_rest(x, t, e_hat, Wn1a, Ws1, bs1, Ws2, bs2, Ws3, bs3):
    out_shapes = (
        jax.ShapeDtypeStruct((N, H), jnp.float32),
        jax.ShapeDtypeStruct((N, 1), jnp.float32),
        jax.ShapeDtypeStruct((N, 1), jnp.float32),
    )
    return pl.pallas_call(_prep_rest_body, out_shape=out_shapes)(
        x, t.reshape(N, 1), e_hat.reshape(N, 1), Wn1a,
        Ws1, bs1.reshape(1, H), Ws2, bs2.reshape(1, H), Ws3, bs3.reshape(1, 1))


def _sc_gather_body(T_hbm, idx_hbm, Tg_hbm, idx_v, rows_v, sem_r):
    wid = lax.axis_index("s") * NC + lax.axis_index("c")
    ppw = idx_hbm.shape[0] // NW
    base = pl.multiple_of(wid * ppw, ppw)
    for c in range(ppw // CH):
        off = pl.multiple_of(base + c * CH, CH)
        pltpu.sync_copy(idx_hbm.at[pl.ds(off, CH)], idx_v)
        pltpu.async_copy(T_hbm.at[idx_v], rows_v, sem_r).wait()
        pltpu.sync_copy(rows_v, Tg_hbm.at[pl.ds(off, CH)])


def _sc_gather(T, chosen_flat):
    npairs = chosen_flat.shape[0]
    mesh = plsc.VectorSubcoreMesh(core_axis_name="c", subcore_axis_name="s")
    kern = pl.kernel(
        _sc_gather_body,
        out_type=jax.ShapeDtypeStruct((npairs, D), jnp.float32),
        mesh=mesh,
        compiler_params=pltpu.CompilerParams(needs_layout_passes=False),
        scratch_types=[
            pltpu.VMEM((CH,), jnp.int32),
            pltpu.VMEM((CH, D), jnp.float32),
            pltpu.SemaphoreType.DMA,
        ],
    )
    return kern(T, chosen_flat)


def _main_body(A_ref, Tg_ref, Wn2_ref, bn2_ref, Wn3_ref, bn3_ref, m_ref):
    RB = A_ref.shape[0]
    h1 = jax.nn.relu(Tg_ref[...][:, :, :H] + A_ref[...][:, None, :])
    h1 = h1.reshape(RB * K, H)
    h2 = jax.nn.relu(jnp.dot(h1, Wn2_ref[...], preferred_element_type=jnp.float32)
                     + bn2_ref[...])
    h2 = h2.reshape(RB, K, H)
    m_ref[...] = (jnp.sum(h2 * Wn3_ref[...].reshape(1, 1, H), axis=2)
                  + bn3_ref[0, 0])


def _tc_main(A, Tg3, Wn2, bn2, Wn3, bn3):
    RB = 512
    grid = (A.shape[0] // RB,)
    return pl.pallas_call(
        _main_body,
        grid=grid,
        in_specs=[
            pl.BlockSpec((RB, H), lambda i: (i, 0)),
            pl.BlockSpec((RB, K, D), lambda i: (i, 0, 0)),
            pl.BlockSpec((H, H), lambda i: (0, 0)),
            pl.BlockSpec((1, H), lambda i: (0, 0)),
            pl.BlockSpec((H, 1), lambda i: (0, 0)),
            pl.BlockSpec((1, 1), lambda i: (0, 0)),
        ],
        out_specs=pl.BlockSpec((RB, K), lambda i: (i, 0)),
        out_shape=jax.ShapeDtypeStruct((A.shape[0], K), jnp.float32),
    )(A, Tg3, Wn2, bn2.reshape(1, H), Wn3, bn3.reshape(1, 1))


def _sc_scatter_body(idx_hbm, m_hbm, r_hbm, sc_hbm, b_hbm, out_hbm, y_hbm,
                     cidx_v, m_v, vals_v, r_v, sc_v, b_v, y_v,
                     buf0, buf1, sem0, sem1):
    wid = lax.axis_index("s") * NC + lax.axis_index("c")
    base = pl.multiple_of(wid * PPW, PPW)
    rbase = pl.multiple_of(wid * ROWS_PW, ROWS_PW)
    pltpu.sync_copy(idx_hbm.at[pl.ds(base, PPW)], cidx_v)
    pltpu.sync_copy(m_hbm.at[pl.ds(base, PPW)], m_v)
    pltpu.sync_copy(r_hbm, r_v)
    pltpu.sync_copy(sc_hbm.at[pl.ds(rbase, ROWS_PW)], sc_v)
    pltpu.sync_copy(b_hbm, b_v)

    z16 = jnp.zeros((L,), jnp.float32)
    b_vec = b_v[...]
    lane = lax.iota(jnp.int32, L)

    def _soft(o):
        acc = z16
        for rr in range(L):
            sl = pl.ds(o * (L * K) + rr * K, L)
            m = m_v[sl]
            cols = cidx_v[sl]
            am = b_vec * jnp.abs(m)
            e = jnp.exp(am - jnp.max(am, axis=0))
            s = lax.broadcast_in_dim(jnp.sum(e, axis=0), (L,), ())
            vals = m * e / s
            prg = plsc.load_gather(r_v, [cols])
            neigh = jnp.sum(prg * vals, axis=0)
            acc = jnp.where(lane == rr, neigh, acc)
            grow = rbase + o * L + rr
            vals_v[sl] = jnp.where(cols == grow, 0.0, vals)
        y_v[pl.ds(o * L, L)] = acc + sc_v[pl.ds(o * L, L)]

    def _zero(i, carry):
        for rr in range(G):
            buf0[rr, pl.ds(i * L, L)] = z16
            buf1[rr, pl.ds(i * L, L)] = z16
        return carry

    lax.fori_loop(0, N // L, _zero, 0)

    bufs = (buf0, buf1)
    sems = (sem0, sem1)
    rowids = [jnp.full((L,), rr, jnp.int32) for rr in range(G)]
    handles = [None] * NGROUPS
    for g in range(NGROUPS):
        if g % 2 == 0:
            _soft(g // 2)
        buf = bufs[g % 2]
        if g >= 2:
            handles[g - 2].wait()
            for rr in range(G):
                cols = cidx_v[pl.ds(((g - 2) * G + rr) * K, L)]
                plsc.store_scatter(buf, [rowids[rr], cols], z16)
        for rr in range(G):
            cols = cidx_v[pl.ds((g * G + rr) * K, L)]
            v = vals_v[pl.ds((g * G + rr) * K, L)]
            plsc.store_scatter(buf, [rowids[rr], cols], v)
        row0 = pl.multiple_of(wid * ROWS_PW + g * G, G)
        handles[g] = pltpu.async_copy(buf, out_hbm.at[pl.ds(row0, G)],
                                      sems[g % 2])
    pltpu.sync_copy(y_v, y_hbm.at[pl.ds(rbase, ROWS_PW)])
    handles[NGROUPS - 2].wait()
    handles[NGROUPS - 1].wait()


def _sc_scatter(chosen_flat, m_flat, r, self_contrib, b_vec):
    mesh = plsc.VectorSubcoreMesh(core_axis_name="c", subcore_axis_name="s")
    kern = pl.kernel(
        _sc_scatter_body,
        out_type=(
            jax.ShapeDtypeStruct((N, N), jnp.float32),
            jax.ShapeDtypeStruct((N,), jnp.float32),
        ),
        mesh=mesh,
        compiler_params=pltpu.CompilerParams(needs_layout_passes=False),
        scratch_types=[
            pltpu.VMEM((PPW,), jnp.int32),
            pltpu.VMEM((PPW,), jnp.float32),
            pltpu.VMEM((PPW,), jnp.float32),
            pltpu.VMEM((N,), jnp.float32),
            pltpu.VMEM((ROWS_PW,), jnp.float32),
            pltpu.VMEM((L,), jnp.float32),
            pltpu.VMEM((ROWS_PW,), jnp.float32),
            pltpu.VMEM((G, N), jnp.float32),
            pltpu.VMEM((G, N), jnp.float32),
            pltpu.SemaphoreType.DMA,
            pltpu.SemaphoreType.DMA,
        ],
    )
    return kern(chosen_flat, m_flat, r, self_contrib, b_vec)


def kernel(x, t, e_hat, nbrs_idx, Wn1, bn1, Wn2, bn2, Wn3, bn3,
           Ws1, bs1, Ws2, bs2, Ws3, bs3, b):
    chosen_flat = nbrs_idx[:, 1:].reshape(PAIRS)
    T = _prep_T(x, t, e_hat, Wn1[D:], bn1)
    Tg = _sc_gather(T, chosen_flat)
    A, g, self_contrib = _tc_prep_rest(
        x, t, e_hat, Wn1[:D], Ws1, bs1, Ws2, bs2, Ws3, bs3)
    m = _tc_main(A, Tg.reshape(N, K, D), Wn2, bn2, Wn3, bn3)
    r = T[:, H]
    pairwise, ypred = _sc_scatter(
        chosen_flat, m.reshape(PAIRS), r, self_contrib.reshape(N),
        jnp.full((L,), b, jnp.float32))
    return ypred, pairwise, g.reshape(N)


# speedup vs baseline: 1.2181x; 1.0076x over previous
import functools

import jax
import jax.numpy as jnp
from jax import lax
from jax.experimental import pallas as pl
from jax.experimental.pallas import tpu as pltpu
from jax.experimental.pallas import tpu_sc as plsc

N = 4096
D = 128
H = 64
K = 16

NC = 2
NS = 16
NW = NC * NS
L = 16

PAIRS = N * K
PPW = PAIRS // NW
CH = 512
ROWS_PW = N // NW
G = 8
NGROUPS = ROWS_PW // G


def _prep_T_body(x_ref, t_ref, e_ref, Wn1b_ref, bn1_ref, T_ref):
    x = x_ref[...]
    Bm = (jnp.dot(x, Wn1b_ref[...], preferred_element_type=jnp.float32)
          + bn1_ref[...])
    r = t_ref[...] - e_ref[...]
    T_ref[...] = jnp.concatenate(
        [Bm, r, jnp.zeros((x.shape[0], D - H - 1), jnp.float32)], axis=1)


def _prep_T(x, t, e_hat, Wn1b, bn1):
    return pl.pallas_call(
        _prep_T_body,
        out_shape=jax.ShapeDtypeStruct((N, D), jnp.float32),
    )(x, t.reshape(N, 1), e_hat.reshape(N, 1), Wn1b, bn1.reshape(1, H))


def _prep_rest_body(x_ref, t_ref, e_ref, Wn1a_ref,
                    Ws1_ref, bs1_ref, Ws2_ref, bs2_ref, Ws3_ref, bs3_ref,
                    A_ref, g_ref, sc_ref):
    x = x_ref[...]
    A_ref[...] = jnp.dot(x, Wn1a_ref[...], preferred_element_type=jnp.float32)
    r = t_ref[...] - e_ref[...]
    h = jax.nn.relu(jnp.dot(x, Ws1_ref[...], preferred_element_type=jnp.float32)
                    + bs1_ref[...])
    h = jax.nn.relu(jnp.dot(h, Ws2_ref[...], preferred_element_type=jnp.float32)
                    + bs2_ref[...])
    g = jnp.sum(h * Ws3_ref[...].reshape(1, H), axis=1, keepdims=True) + bs3_ref[0, 0]
    g_ref[...] = g
    sc_ref[...] = g * r


def _tc_prep_rest(x, t, e_hat, Wn1a, Ws1, bs1, Ws2, bs2, Ws3, bs3):
    out_shapes = (
        jax.ShapeDtypeStruct((N, H), jnp.float32),
        jax.ShapeDtypeStruct((N, 1), jnp.float32),
        jax.ShapeDtypeStruct((N, 1), jnp.float32),
    )
    return pl.pallas_call(_prep_rest_body, out_shape=out_shapes)(
        x, t.reshape(N, 1), e_hat.reshape(N, 1), Wn1a,
        Ws1, bs1.reshape(1, H), Ws2, bs2.reshape(1, H), Ws3, bs3.reshape(1, 1))


def _sc_gather_body(T_hbm, idx_hbm, Tg_hbm, idx_v, rows_v, sem_r):
    wid = lax.axis_index("s") * NC + lax.axis_index("c")
    ppw = idx_hbm.shape[0] // NW
    base = pl.multiple_of(wid * ppw, ppw)
    for c in range(ppw // CH):
        off = pl.multiple_of(base + c * CH, CH)
        pltpu.sync_copy(idx_hbm.at[pl.ds(off, CH)], idx_v)
        pltpu.async_copy(T_hbm.at[idx_v], rows_v, sem_r).wait()
        pltpu.sync_copy(rows_v, Tg_hbm.at[pl.ds(off, CH)])


def _sc_gather(T, chosen_flat):
    npairs = chosen_flat.shape[0]
    mesh = plsc.VectorSubcoreMesh(core_axis_name="c", subcore_axis_name="s")
    kern = pl.kernel(
        _sc_gather_body,
        out_type=jax.ShapeDtypeStruct((npairs, D), jnp.float32),
        mesh=mesh,
        compiler_params=pltpu.CompilerParams(needs_layout_passes=False),
        scratch_types=[
            pltpu.VMEM((CH,), jnp.int32),
            pltpu.VMEM((CH, D), jnp.float32),
            pltpu.SemaphoreType.DMA,
        ],
    )
    return kern(T, chosen_flat)


def _main_body(A_ref, Tg_ref, Wn2_ref, bn2_ref, Wn3_ref, bn3_ref, m_ref):
    RB = A_ref.shape[0]
    h1 = jax.nn.relu(Tg_ref[...][:, :, :H] + A_ref[...][:, None, :])
    h1 = h1.reshape(RB * K, H)
    h2 = jax.nn.relu(jnp.dot(h1, Wn2_ref[...], preferred_element_type=jnp.float32)
                     + bn2_ref[...])
    h2 = h2.reshape(RB, K, H)
    m_ref[...] = (jnp.sum(h2 * Wn3_ref[...].reshape(1, 1, H), axis=2)
                  + bn3_ref[0, 0])


def _tc_main(A, Tg3, Wn2, bn2, Wn3, bn3):
    RB = 1024
    grid = (A.shape[0] // RB,)
    return pl.pallas_call(
        _main_body,
        grid=grid,
        in_specs=[
            pl.BlockSpec((RB, H), lambda i: (i, 0)),
            pl.BlockSpec((RB, K, D), lambda i: (i, 0, 0)),
            pl.BlockSpec((H, H), lambda i: (0, 0)),
            pl.BlockSpec((1, H), lambda i: (0, 0)),
            pl.BlockSpec((H, 1), lambda i: (0, 0)),
            pl.BlockSpec((1, 1), lambda i: (0, 0)),
        ],
        out_specs=pl.BlockSpec((RB, K), lambda i: (i, 0)),
        out_shape=jax.ShapeDtypeStruct((A.shape[0], K), jnp.float32),
    )(A, Tg3, Wn2, bn2.reshape(1, H), Wn3, bn3.reshape(1, 1))


def _sc_scatter_body(idx_hbm, m_hbm, r_hbm, sc_hbm, b_hbm, out_hbm, y_hbm,
                     cidx_v, m_v, vals_v, r_v, sc_v, b_v, y_v,
                     buf0, buf1, sem0, sem1):
    wid = lax.axis_index("s") * NC + lax.axis_index("c")
    base = pl.multiple_of(wid * PPW, PPW)
    rbase = pl.multiple_of(wid * ROWS_PW, ROWS_PW)
    pltpu.sync_copy(idx_hbm.at[pl.ds(base, PPW)], cidx_v)
    pltpu.sync_copy(m_hbm.at[pl.ds(base, PPW)], m_v)
    pltpu.sync_copy(r_hbm, r_v)
    pltpu.sync_copy(sc_hbm.at[pl.ds(rbase, ROWS_PW)], sc_v)
    pltpu.sync_copy(b_hbm, b_v)

    z16 = jnp.zeros((L,), jnp.float32)
    b_vec = b_v[...]
    lane = lax.iota(jnp.int32, L)

    def _soft(o):
        acc = z16
        for rr in range(L):
            sl = pl.ds(o * (L * K) + rr * K, L)
            m = m_v[sl]
            cols = cidx_v[sl]
            am = b_vec * jnp.abs(m)
            e = jnp.exp(am - jnp.max(am, axis=0))
            s = lax.broadcast_in_dim(jnp.sum(e, axis=0), (L,), ())
            vals = m * e / s
            prg = plsc.load_gather(r_v, [cols])
            neigh = jnp.sum(prg * vals, axis=0)
            acc = jnp.where(lane == rr, neigh, acc)
            grow = rbase + o * L + rr
            vals_v[sl] = jnp.where(cols == grow, 0.0, vals)
        y_v[pl.ds(o * L, L)] = acc + sc_v[pl.ds(o * L, L)]

    def _zero(i, carry):
        for rr in range(G):
            buf0[rr, pl.ds(i * L, L)] = z16
            buf1[rr, pl.ds(i * L, L)] = z16
        return carry

    lax.fori_loop(0, N // L, _zero, 0)

    bufs = (buf0, buf1)
    sems = (sem0, sem1)
    rowids = [jnp.full((L,), rr, jnp.int32) for rr in range(G)]
    handles = [None] * NGROUPS
    for g in range(NGROUPS):
        if g % 2 == 0:
            _soft(g // 2)
        buf = bufs[g % 2]
        if g >= 2:
            handles[g - 2].wait()
            for rr in range(G):
                cols = cidx_v[pl.ds(((g - 2) * G + rr) * K, L)]
                plsc.store_scatter(buf, [rowids[rr], cols], z16)
        for rr in range(G):
            cols = cidx_v[pl.ds((g * G + rr) * K, L)]
            v = vals_v[pl.ds((g * G + rr) * K, L)]
            plsc.store_scatter(buf, [rowids[rr], cols], v)
        row0 = pl.multiple_of(wid * ROWS_PW + g * G, G)
        handles[g] = pltpu.async_copy(buf, out_hbm.at[pl.ds(row0, G)],
                                      sems[g % 2])
    pltpu.sync_copy(y_v, y_hbm.at[pl.ds(rbase, ROWS_PW)])
    handles[NGROUPS - 2].wait()
    handles[NGROUPS - 1].wait()


def _sc_scatter(chosen_flat, m_flat, r, self_contrib, b_vec):
    mesh = plsc.VectorSubcoreMesh(core_axis_name="c", subcore_axis_name="s")
    kern = pl.kernel(
        _sc_scatter_body,
        out_type=(
            jax.ShapeDtypeStruct((N, N), jnp.float32),
            jax.ShapeDtypeStruct((N,), jnp.float32),
        ),
        mesh=mesh,
        compiler_params=pltpu.CompilerParams(needs_layout_passes=False),
        scratch_types=[
            pltpu.VMEM((PPW,), jnp.int32),
            pltpu.VMEM((PPW,), jnp.float32),
            pltpu.VMEM((PPW,), jnp.float32),
            pltpu.VMEM((N,), jnp.float32),
            pltpu.VMEM((ROWS_PW,), jnp.float32),
            pltpu.VMEM((L,), jnp.float32),
            pltpu.VMEM((ROWS_PW,), jnp.float32),
            pltpu.VMEM((G, N), jnp.float32),
            pltpu.VMEM((G, N), jnp.float32),
            pltpu.SemaphoreType.DMA,
            pltpu.SemaphoreType.DMA,
        ],
    )
    return kern(chosen_flat, m_flat, r, self_contrib, b_vec)


def kernel(x, t, e_hat, nbrs_idx, Wn1, bn1, Wn2, bn2, Wn3, bn3,
           Ws1, bs1, Ws2, bs2, Ws3, bs3, b):
    chosen_flat = nbrs_idx[:, 1:].reshape(PAIRS)
    T = _prep_T(x, t, e_hat, Wn1[D:], bn1)
    Tg = _sc_gather(T, chosen_flat)
    A, g, self_contrib = _tc_prep_rest(
        x, t, e_hat, Wn1[:D], Ws1, bs1, Ws2, bs2, Ws3, bs3)
    m = _tc_main(A, Tg.reshape(N, K, D), Wn2, bn2, Wn3, bn3)
    r = T[:, H]
    pairwise, ypred = _sc_scatter(
        chosen_flat, m.reshape(PAIRS), r, self_contrib.reshape(N),
        jnp.full((L,), b, jnp.float32))
    return ypred, pairwise, g.reshape(N)
